# baseline (speedup 1.0000x reference)
import jax
import jax.numpy as jnp
from jax import lax
from jax.experimental import pallas as pl
from jax.experimental.pallas import tpu as pltpu

_BLK = 8192
_TCH = 256


def _router_body(x_ref, w1_ref, b1_ref, w2_ref, b2_ref, eye_ref, wt_ref, idxt_ref):
    x = x_ref[...]
    h = jnp.maximum(
        jnp.dot(x, w1_ref[...], preferred_element_type=jnp.float32) + b1_ref[...],
        0.0,
    )
    logits = jnp.dot(h, w2_ref[...], preferred_element_type=jnp.float32) + b2_ref[...]
    E = logits.shape[1]
    lane = lax.broadcasted_iota(jnp.int32, logits.shape, 1).astype(jnp.float32)
    m1 = jnp.max(logits, axis=1, keepdims=True)
    idx1 = jnp.min(jnp.where(logits == m1, lane, float(E)), axis=1, keepdims=True)
    rest = jnp.where(lane == idx1, -jnp.inf, logits)
    m2 = jnp.max(rest, axis=1, keepdims=True)
    idx2 = jnp.min(jnp.where(rest == m2, lane, float(E)), axis=1, keepdims=True)
    e2 = jnp.exp(m2 - m1)
    p2 = e2 / (1.0 + e2)
    p1 = 1.0 - p2
    p1h = p1.astype(jnp.bfloat16).astype(jnp.float32)
    p1l = (p1 - p1h).astype(jnp.bfloat16).astype(jnp.float32)
    p2h = p2.astype(jnp.bfloat16).astype(jnp.float32)
    p2l = (p2 - p2h).astype(jnp.bfloat16).astype(jnp.float32)
    zero = jnp.zeros_like(p1)
    small = jnp.concatenate(
        [idx1, idx2, p1h, p1l, p2h, p2l, zero, zero], axis=1
    )
    eye = eye_ref[...]
    pieces = [
        lax.dot_general(
            small[j * _TCH:(j + 1) * _TCH, :],
            eye,
            ((( 0,), (0,)), ((), ())),
            preferred_element_type=jnp.float32,
        )
        for j in range(_BLK // _TCH)
    ]
    small_t = jnp.concatenate(pieces, axis=1)
    idxt_ref[...] = small_t.astype(jnp.int32)
    i1r = small_t[0:1, :]
    i2r = small_t[1:2, :]
    p1r = small_t[2:3, :] + small_t[3:4, :]
    p2r = small_t[4:5, :] + small_t[5:6, :]
    sub = lax.broadcasted_iota(jnp.int32, (E, small.shape[0]), 0).astype(jnp.float32)
    wt_ref[...] = jnp.where(sub == i1r, p1r, jnp.where(sub == i2r, p2r, 0.0))


def kernel(x, top_k, W1, b1, W2, b2):
    del top_k
    B, D = x.shape
    H = W1.shape[1]
    E = W2.shape[1]
    grid = (B // _BLK,)
    eye = jnp.eye(_TCH, dtype=jnp.float32)
    wt, idxt = pl.pallas_call(
        _router_body,
        grid=grid,
        in_specs=[
            pl.BlockSpec((_BLK, D), lambda i: (i, 0)),
            pl.BlockSpec((D, H), lambda i: (0, 0)),
            pl.BlockSpec((1, H), lambda i: (0, 0)),
            pl.BlockSpec((H, E), lambda i: (0, 0)),
            pl.BlockSpec((1, E), lambda i: (0, 0)),
            pl.BlockSpec((_TCH, _TCH), lambda i: (0, 0)),
        ],
        out_specs=[
            pl.BlockSpec((E, _BLK), lambda i: (0, i)),
            pl.BlockSpec((8, _BLK), lambda i: (0, i)),
        ],
        out_shape=[
            jax.ShapeDtypeStruct((E, B), jnp.float32),
            jax.ShapeDtypeStruct((8, B), jnp.int32),
        ],
        compiler_params=pltpu.CompilerParams(
            dimension_semantics=("parallel",),
        ),
    )(x, W1, b1.reshape(1, H), W2, b2.reshape(1, E), eye)
    return wt.T, idxt[:2, :].T


# speedup vs baseline: 1.0372x; 1.0372x over previous
import jax
import jax.numpy as jnp
from jax import lax
from jax.experimental import pallas as pl
from jax.experimental.pallas import tpu as pltpu

_BLK = 4096
_TCH = 256


def _router_body(x_ref, w1_ref, b1_ref, w2_ref, b2_ref, eye_ref, wt_ref, idxt_ref):
    x = x_ref[...]
    h = jnp.maximum(
        jnp.dot(x, w1_ref[...], preferred_element_type=jnp.float32) + b1_ref[...],
        0.0,
    )
    logits = jnp.dot(h, w2_ref[...], preferred_element_type=jnp.float32) + b2_ref[...]
    E = logits.shape[1]
    lane = lax.broadcasted_iota(jnp.int32, logits.shape, 1).astype(jnp.float32)
    m1 = jnp.max(logits, axis=1, keepdims=True)
    idx1 = jnp.min(jnp.where(logits == m1, lane, float(E)), axis=1, keepdims=True)
    rest = jnp.where(lane == idx1, -jnp.inf, logits)
    m2 = jnp.max(rest, axis=1, keepdims=True)
    idx2 = jnp.min(jnp.where(rest == m2, lane, float(E)), axis=1, keepdims=True)
    e2 = jnp.exp(m2 - m1)
    p2 = e2 / (1.0 + e2)
    p1 = 1.0 - p2
    p1h = p1.astype(jnp.bfloat16).astype(jnp.float32)
    p1l = (p1 - p1h).astype(jnp.bfloat16).astype(jnp.float32)
    p2h = p2.astype(jnp.bfloat16).astype(jnp.float32)
    p2l = (p2 - p2h).astype(jnp.bfloat16).astype(jnp.float32)
    zero = jnp.zeros_like(p1)
    small = jnp.concatenate(
        [idx1, idx2, p1h, p1l, p2h, p2l, zero, zero], axis=1
    )
    eye = eye_ref[...]
    pieces = [
        lax.dot_general(
            small[j * _TCH:(j + 1) * _TCH, :],
            eye,
            ((( 0,), (0,)), ((), ())),
            preferred_element_type=jnp.float32,
        )
        for j in range(_BLK // _TCH)
    ]
    small_t = jnp.concatenate(pieces, axis=1)
    idxt_ref[...] = small_t.astype(jnp.int32)
    i1r = small_t[0:1, :]
    i2r = small_t[1:2, :]
    p1r = small_t[2:3, :] + small_t[3:4, :]
    p2r = small_t[4:5, :] + small_t[5:6, :]
    sub = lax.broadcasted_iota(jnp.int32, (E, small.shape[0]), 0).astype(jnp.float32)
    wt_ref[...] = jnp.where(sub == i1r, p1r, jnp.where(sub == i2r, p2r, 0.0))


def kernel(x, top_k, W1, b1, W2, b2):
    del top_k
    B, D = x.shape
    H = W1.shape[1]
    E = W2.shape[1]
    grid = (B // _BLK,)
    eye = jnp.eye(_TCH, dtype=jnp.float32)
    wt, idxt = pl.pallas_call(
        _router_body,
        grid=grid,
        in_specs=[
            pl.BlockSpec((_BLK, D), lambda i: (i, 0)),
            pl.BlockSpec((D, H), lambda i: (0, 0)),
            pl.BlockSpec((1, H), lambda i: (0, 0)),
            pl.BlockSpec((H, E), lambda i: (0, 0)),
            pl.BlockSpec((1, E), lambda i: (0, 0)),
            pl.BlockSpec((_TCH, _TCH), lambda i: (0, 0)),
        ],
        out_specs=[
            pl.BlockSpec((E, _BLK), lambda i: (0, i)),
            pl.BlockSpec((8, _BLK), lambda i: (0, i)),
        ],
        out_shape=[
            jax.ShapeDtypeStruct((E, B), jnp.float32),
            jax.ShapeDtypeStruct((8, B), jnp.int32),
        ],
        compiler_params=pltpu.CompilerParams(
            dimension_semantics=("parallel",),
        ),
    )(x, W1, b1.reshape(1, H), W2, b2.reshape(1, E), eye)
    return wt.T, idxt[:2, :].T
